# baseline (speedup 1.0000x reference)
import jax
import jax.numpy as jnp
from jax.experimental import pallas as pl
from jax.experimental.pallas import tpu as pltpu

B = 64
V = 100000
D = 128
MAX_SEQ = 2048
CTX = 1
MAX_GEN = MAX_SEQ - CTX
PAD = 0
EOS = 2

SUB = 8192
NCH = (V + SUB - 1) // SUB
VPADDED = NCH * SUB
NEG = -1e30


def _gather_body(s_ref, emb_ref, h_out, sem):
    def _start(r, c):
        idx = s_ref[r]
        pltpu.make_async_copy(
            emb_ref.at[pl.ds(idx, 1), :], h_out.at[pl.ds(r, 1), :], sem
        ).start()
        return c

    jax.lax.fori_loop(0, B, _start, 0)

    def _wait(r, c):
        idx = s_ref[r]
        pltpu.make_async_copy(
            emb_ref.at[pl.ds(idx, 1), :], h_out.at[pl.ds(r, 1), :], sem
        ).wait()
        return c

    jax.lax.fori_loop(0, B, _wait, 0)


def _partial_body(h_ref, w_ref, b_ref, pv_out, pi_out):
    i = pl.program_id(0)
    logits = jnp.dot(h_ref[:], w_ref[:], preferred_element_type=jnp.float32)
    logits = logits + b_ref[0, :][None, :]
    col_ids = i * SUB + jax.lax.broadcasted_iota(jnp.int32, (1, SUB), 1)
    logits = jnp.where(col_ids < V, logits, NEG)
    cmax = jnp.max(logits, axis=1, keepdims=True)
    carg = jnp.min(jnp.where(logits == cmax, col_ids, V), axis=1,
                   keepdims=True)
    pv_out[:] = jnp.broadcast_to(cmax, (B, 128))
    pi_out[:] = jnp.broadcast_to(carg.astype(jnp.int32), (B, 128))


def _merge_body(s_ref, cur_vec_ref, pv_ref, pi_ref, gen_ref,
                tok_out, buf_out, step_out):
    pv = pv_ref[:]
    pi = pi_ref[:]
    m = jnp.max(pv, axis=1, keepdims=True)
    bi = jnp.min(jnp.where(pv == m, pi, V), axis=1, keepdims=True)
    cur = cur_vec_ref[:]
    tok = jnp.where(cur == EOS, EOS, bi.astype(jnp.int32))
    tok_out[:] = tok
    col = s_ref[0]
    begin_new = s_ref[1]
    keep = 1.0 - begin_new.astype(jnp.float32)
    base_buf = gen_ref[:] * keep + (1.0 - keep) * jnp.float32(PAD)
    cids = jax.lax.broadcasted_iota(jnp.int32, (B, MAX_GEN), 1)
    add = jnp.where(cids == col, tok.astype(jnp.float32) - jnp.float32(PAD), 0.0)
    buf_out[:] = base_buf + add
    step_out[0] = col.astype(jnp.float32) + 1.0


def kernel(decoder_input_ids, emb, W_out, b_out, generated_tokens, generation_step):
    stepf = generation_step[0]
    stepc = jnp.where(stepf < MAX_GEN, stepf, 0.0)
    begin_new = (stepc == 0.0).astype(jnp.int32)
    col = stepc.astype(jnp.int32)
    prev_col = jnp.maximum(col - 1, 0)
    prev = jax.lax.dynamic_slice(generated_tokens, (0, prev_col), (B, 1))
    cur = jnp.where(begin_new == 1, decoder_input_ids[:, 0],
                    prev[:, 0].astype(jnp.int32))
    cur_vec = cur[:, None]
    b2 = jnp.pad(b_out.reshape(1, V), ((0, 0), (0, VPADDED - V)),
                 constant_values=NEG)

    h = pl.pallas_call(
        _gather_body,
        grid_spec=pltpu.PrefetchScalarGridSpec(
            num_scalar_prefetch=1,
            grid=(1,),
            in_specs=[pl.BlockSpec(memory_space=pltpu.HBM)],
            out_specs=pl.BlockSpec((B, D), lambda i, s: (0, 0)),
            scratch_shapes=[pltpu.SemaphoreType.DMA],
        ),
        out_shape=jax.ShapeDtypeStruct((B, D), jnp.float32),
        compiler_params=pltpu.CompilerParams(
            dimension_semantics=("arbitrary",),
        ),
    )(cur, emb)

    pv, pi = pl.pallas_call(
        _partial_body,
        grid=(NCH,),
        in_specs=[
            pl.BlockSpec((B, D), lambda i: (0, 0)),
            pl.BlockSpec((D, SUB), lambda i: (0, i)),
            pl.BlockSpec((1, SUB), lambda i: (0, i)),
        ],
        out_specs=[
            pl.BlockSpec((B, 128), lambda i: (0, i)),
            pl.BlockSpec((B, 128), lambda i: (0, i)),
        ],
        out_shape=[
            jax.ShapeDtypeStruct((B, NCH * 128), jnp.float32),
            jax.ShapeDtypeStruct((B, NCH * 128), jnp.int32),
        ],
        compiler_params=pltpu.CompilerParams(
            dimension_semantics=("parallel",),
        ),
    )(h, W_out, b2)

    scalars = jnp.stack([col, begin_new])
    tokens, new_buffer, new_step = pl.pallas_call(
        _merge_body,
        grid_spec=pltpu.PrefetchScalarGridSpec(
            num_scalar_prefetch=1,
            grid=(1,),
            in_specs=[
                pl.BlockSpec((B, 1), lambda i, s: (0, 0)),
                pl.BlockSpec((B, NCH * 128), lambda i, s: (0, 0)),
                pl.BlockSpec((B, NCH * 128), lambda i, s: (0, 0)),
                pl.BlockSpec((B, MAX_GEN), lambda i, s: (0, 0)),
            ],
            out_specs=[
                pl.BlockSpec((B, 1), lambda i, s: (0, 0)),
                pl.BlockSpec((B, MAX_GEN), lambda i, s: (0, 0)),
                pl.BlockSpec(memory_space=pltpu.SMEM),
            ],
        ),
        out_shape=[
            jax.ShapeDtypeStruct((B, 1), jnp.int32),
            jax.ShapeDtypeStruct((B, MAX_GEN), jnp.float32),
            jax.ShapeDtypeStruct((1,), jnp.float32),
        ],
        compiler_params=pltpu.CompilerParams(
            dimension_semantics=("arbitrary",),
        ),
    )(scalars, cur_vec, pv, pi, generated_tokens)
    return tokens, new_buffer, new_step
